# baseline (speedup 1.0000x reference)
import functools

import jax
import jax.numpy as jnp
from jax import lax
from jax.experimental import pallas as pl
from jax.experimental.pallas import tpu as pltpu
from jax.experimental.pallas import tpu_sc as plsc

D = 64
SEQ = 200
LANES = 16
BBLK = 128
NBUF = 4


@jax.jit
def kernel(inputs, token_table, pos_table):
    B, S = inputs.shape
    assert S == SEQ and token_table.shape[1] == D
    idx_t = inputs.T

    info = plsc.get_sparse_core_info()
    nw = info.num_cores * info.num_subcores
    assert B == nw * BBLK

    mesh = plsc.VectorSubcoreMesh(core_axis_name="c", subcore_axis_name="s")

    @functools.partial(
        pl.kernel,
        mesh=mesh,
        out_type=jax.ShapeDtypeStruct((SEQ, D // 8, B // BBLK, 8, BBLK),
                                      jnp.float32),
        compiler_params=pltpu.CompilerParams(use_tc_tiling_on_sc=False,
                                             needs_layout_passes=False),
        scratch_types=[
            pltpu.VMEM((SEQ, BBLK), jnp.int32),
            pltpu.VMEM((SEQ, D), jnp.float32),
            pltpu.VMEM((NBUF, BBLK, D), jnp.float32),
            pltpu.VMEM((NBUF, D // 8, 8, BBLK + 1), jnp.float32),
        ] + [pltpu.SemaphoreType.DMA] * (2 * NBUF),
    )
    def sc_embed(idx_hbm, table_hbm, pos_hbm, out_hbm,
                 idx_v, pos_v, rows_v, patch_v, *sems):
        gsem = sems[:NBUF]
        ssem = sems[NBUF:]
        wid = lax.axis_index("s") * info.num_cores + lax.axis_index("c")
        pltpu.sync_copy(idx_hbm.at[:, pl.ds(wid * BBLK, BBLK)], idx_v)
        pltpu.sync_copy(pos_hbm, pos_v)

        iota = jax.lax.iota(jnp.int32, LANES)
        idx_dt = [(iota + 16 * j) >> 3 for j in range(D // LANES)]
        idx_d8 = [(iota + 16 * j) & 7 for j in range(D // LANES)]

        def gather_copy(s, slot):
            return pltpu.make_async_copy(
                table_hbm.at[idx_v.at[s]], rows_v.at[slot], gsem[slot])

        def store_copies(s, slot):
            return [pltpu.make_async_copy(
                patch_v.at[slot].at[:, :, pl.ds(0, BBLK)],
                out_hbm.at[s, :, wid], ssem[slot])]

        gather_copy(0, 0).start()
        gather_copy(1, 1).start()

        def quad(i, carry):
            for k in range(NBUF):
                s = i * NBUF + k
                rows = rows_v.at[k]
                patch = patch_v.at[k]
                gather_copy(s, k).wait()

                @pl.when(s + 2 < SEQ)
                def _():
                    gather_copy(s + 2, (k + 2) % NBUF).start()

                @pl.when(s >= NBUF)
                def _():
                    for cp in store_copies(0, k):
                        cp.wait()

                pv = [pos_v[s, pl.ds(LANES * j, LANES)]
                      for j in range(D // LANES)]

                @plsc.parallel_loop(0, BBLK, unroll=4)
                def _(b):
                    ib = jnp.full((LANES,), b, jnp.int32)
                    for j in range(D // LANES):
                        x = rows[b, pl.ds(LANES * j, LANES)] + pv[j]
                        plsc.store_scatter(
                            patch, [idx_dt[j], idx_d8[j], ib], x)
                for cp in store_copies(s, k):
                    cp.start()
            return carry

        lax.fori_loop(0, SEQ // NBUF, quad, 0)
        for k in range(NBUF):
            for cp in store_copies(0, k):
                cp.wait()

    out5 = sc_embed(idx_t, token_table, pos_table)
    return out5.transpose(2, 4, 0, 1, 3).reshape(B, SEQ, D)


# speedup vs baseline: 1.0408x; 1.0408x over previous
import functools

import jax
import jax.numpy as jnp
from jax import lax
from jax.experimental import pallas as pl
from jax.experimental.pallas import tpu as pltpu
from jax.experimental.pallas import tpu_sc as plsc

D = 64
SEQ = 200
LANES = 16
BBLK = 128
NBUF = 4


@jax.jit
def kernel(inputs, token_table, pos_table):
    B, S = inputs.shape
    assert S == SEQ and token_table.shape[1] == D
    idx_t = inputs.T

    info = plsc.get_sparse_core_info()
    nw = info.num_cores * info.num_subcores
    assert B == nw * BBLK

    mesh = plsc.VectorSubcoreMesh(core_axis_name="c", subcore_axis_name="s")

    @functools.partial(
        pl.kernel,
        mesh=mesh,
        out_type=jax.ShapeDtypeStruct((SEQ, D // 8, B // BBLK, 8, BBLK),
                                      jnp.float32),
        compiler_params=pltpu.CompilerParams(use_tc_tiling_on_sc=False,
                                             needs_layout_passes=False),
        scratch_types=[
            pltpu.VMEM((SEQ, BBLK), jnp.int32),
            pltpu.VMEM((SEQ, D), jnp.float32),
            pltpu.VMEM((NBUF, BBLK, D), jnp.float32),
            pltpu.VMEM((NBUF, D // 8, 8, BBLK + 1), jnp.float32),
        ] + [pltpu.SemaphoreType.DMA] * (2 * NBUF),
    )
    def sc_embed(idx_hbm, table_hbm, pos_hbm, out_hbm,
                 idx_v, pos_v, rows_v, patch_v, *sems):
        gsem = sems[:NBUF]
        ssem = sems[NBUF:]
        wid = lax.axis_index("s") * info.num_cores + lax.axis_index("c")
        pltpu.sync_copy(idx_hbm.at[:, pl.ds(wid * BBLK, BBLK)], idx_v)
        pltpu.sync_copy(pos_hbm, pos_v)

        iota = jax.lax.iota(jnp.int32, LANES)
        idx_dt = [(iota + 16 * j) >> 3 for j in range(D // LANES)]
        idx_d8 = [(iota + 16 * j) & 7 for j in range(D // LANES)]

        def gather_copy(s, slot):
            return pltpu.make_async_copy(
                table_hbm.at[idx_v.at[s]], rows_v.at[slot], gsem[slot])

        def store_copies(s, slot):
            return [pltpu.make_async_copy(
                patch_v.at[slot].at[:, :, pl.ds(0, BBLK)],
                out_hbm.at[s, :, wid], ssem[slot])]

        for g0 in range(3):
            gather_copy(g0, g0).start()

        def quad(i, carry):
            for k in range(NBUF):
                s = i * NBUF + k
                rows = rows_v.at[k]
                patch = patch_v.at[k]
                gather_copy(s, k).wait()

                @pl.when(s + 3 < SEQ)
                def _():
                    gather_copy(s + 3, (k + 3) % NBUF).start()

                @pl.when(s >= NBUF)
                def _():
                    for cp in store_copies(0, k):
                        cp.wait()

                pv = [pos_v[s, pl.ds(LANES * j, LANES)]
                      for j in range(D // LANES)]

                @plsc.parallel_loop(0, BBLK, unroll=4)
                def _(b):
                    ib = jnp.full((LANES,), b, jnp.int32)
                    for j in range(D // LANES):
                        x = rows[b, pl.ds(LANES * j, LANES)] + pv[j]
                        plsc.store_scatter(
                            patch, [idx_dt[j], idx_d8[j], ib], x)
                for cp in store_copies(s, k):
                    cp.start()
            return carry

        lax.fori_loop(0, SEQ // NBUF, quad, 0)
        for k in range(NBUF):
            for cp in store_copies(0, k):
                cp.wait()

    out5 = sc_embed(idx_t, token_table, pos_table)
    return out5.transpose(2, 4, 0, 1, 3).reshape(B, SEQ, D)


# speedup vs baseline: 1.0448x; 1.0038x over previous
import functools

import jax
import jax.numpy as jnp
from jax import lax
from jax.experimental import pallas as pl
from jax.experimental.pallas import tpu as pltpu
from jax.experimental.pallas import tpu_sc as plsc

D = 64
SEQ = 200
LANES = 16
BBLK = 128
NBUF = 4


@jax.jit
def kernel(inputs, token_table, pos_table):
    B, S = inputs.shape
    assert S == SEQ and token_table.shape[1] == D
    idx_t = inputs.T

    info = plsc.get_sparse_core_info()
    nw = info.num_cores * info.num_subcores
    assert B == nw * BBLK

    mesh = plsc.VectorSubcoreMesh(core_axis_name="c", subcore_axis_name="s")

    @functools.partial(
        pl.kernel,
        mesh=mesh,
        out_type=jax.ShapeDtypeStruct((SEQ, D // 8, B // BBLK, 8, BBLK),
                                      jnp.float32),
        compiler_params=pltpu.CompilerParams(use_tc_tiling_on_sc=False,
                                             needs_layout_passes=False),
        scratch_types=[
            pltpu.VMEM((SEQ, BBLK), jnp.int32),
            pltpu.VMEM((SEQ, D), jnp.float32),
            pltpu.VMEM((NBUF, BBLK, D), jnp.float32),
            pltpu.VMEM((NBUF, D // 8, 8, BBLK + 1), jnp.float32),
        ] + [pltpu.SemaphoreType.DMA] * (2 * NBUF),
    )
    def sc_embed(idx_hbm, table_hbm, pos_hbm, out_hbm,
                 idx_v, pos_v, rows_v, patch_v, *sems):
        gsem = sems[:NBUF]
        ssem = sems[NBUF:]
        wid = lax.axis_index("s") * info.num_cores + lax.axis_index("c")
        pltpu.sync_copy(idx_hbm.at[:, pl.ds(wid * BBLK, BBLK)], idx_v)
        pltpu.sync_copy(pos_hbm, pos_v)

        iota = jax.lax.iota(jnp.int32, LANES)
        idx_dt = [(iota + 16 * j) >> 3 for j in range(D // LANES)]
        idx_d8 = [(iota + 16 * j) & 7 for j in range(D // LANES)]

        def gather_copy(s, slot):
            return pltpu.make_async_copy(
                table_hbm.at[idx_v.at[s]], rows_v.at[slot], gsem[slot])

        def store_copies(s, slot):
            return [pltpu.make_async_copy(
                patch_v.at[slot].at[:, :, pl.ds(0, BBLK)],
                out_hbm.at[s, :, wid], ssem[slot])]

        for g0 in range(NBUF):
            gather_copy(g0, g0).start()

        def quad(i, carry):
            for k in range(NBUF):
                s = i * NBUF + k
                rows = rows_v.at[k]
                patch = patch_v.at[k]
                gather_copy(s, k).wait()

                @pl.when(s >= NBUF)
                def _():
                    for cp in store_copies(0, k):
                        cp.wait()

                pv = [pos_v[s, pl.ds(LANES * j, LANES)]
                      for j in range(D // LANES)]

                @plsc.parallel_loop(0, BBLK, unroll=4)
                def _(b):
                    ib = jnp.full((LANES,), b, jnp.int32)
                    for j in range(D // LANES):
                        x = rows[b, pl.ds(LANES * j, LANES)] + pv[j]
                        plsc.store_scatter(
                            patch, [idx_dt[j], idx_d8[j], ib], x)

                @pl.when(s + NBUF < SEQ)
                def _():
                    gather_copy(s + NBUF, k).start()

                for cp in store_copies(s, k):
                    cp.start()
            return carry

        lax.fori_loop(0, SEQ // NBUF, quad, 0)
        for k in range(NBUF):
            for cp in store_copies(0, k):
                cp.wait()

    out5 = sc_embed(idx_t, token_table, pos_table)
    return out5.transpose(2, 4, 0, 1, 3).reshape(B, SEQ, D)
